# baseline (speedup 1.0000x reference)
import functools

import jax
import jax.numpy as jnp
from jax import lax
from jax.experimental import pallas as pl
from jax.experimental.pallas import tpu as pltpu
from jax.experimental.pallas import tpu_sc as plsc

NC = 2
NS = 16
NW = NC * NS
LANES = 16

EMB = 64
E_SL = EMB // LANES
CHUNK = 64


def _region_encode(seq3d, W, U, *, B, L, R):
    TOK = B * L
    b_per_w = B // NW
    per_w = TOK // NW
    n_chunks = per_w // CHUNK
    RAD = (R - 1) // 2

    mesh = plsc.VectorSubcoreMesh(
        core_axis_name="c", subcore_axis_name="s", num_cores=NC, num_subcores=NS
    )

    @functools.partial(
        pl.kernel,
        out_type=jax.ShapeDtypeStruct((TOK, EMB), jnp.float32),
        mesh=mesh,
        compiler_params=pltpu.CompilerParams(
            needs_layout_passes=False, use_tc_tiling_on_sc=False
        ),
        scratch_types=[
            pltpu.VMEM((b_per_w, L, 1), jnp.int32),
            pltpu.VMEM((2, CHUNK), jnp.int32),
            pltpu.VMEM((2, R, CHUNK), jnp.int32),
            pltpu.VMEM((2, CHUNK, EMB), jnp.float32),
            pltpu.VMEM((2, R, CHUNK, EMB), jnp.float32),
            pltpu.VMEM((CHUNK, EMB), jnp.float32),
            pltpu.SemaphoreType.DMA,
            pltpu.SemaphoreType.DMA,
            pltpu.SemaphoreType.DMA,
            pltpu.SemaphoreType.DMA,
        ],
    )
    def k(seq_hbm, W_hbm, U_hbm, out_hbm,
          seq_v, w_idx, u_idx, w_rows, u_rows, out_v, semw0, semw1, semu0, semu1):
        wid = lax.axis_index("s") * NC + lax.axis_index("c")
        base = wid * per_w
        pltpu.sync_copy(seq_hbm.at[pl.ds(wid * b_per_w, b_per_w)], seq_v)

        lane = lax.broadcasted_iota(jnp.int32, (LANES,), 0)
        zero16 = jnp.zeros((LANES,), jnp.int32)
        semw = (semw0, semw1)
        semu = (semu0, semu1)

        def build_and_fire(c, p):
            for j in range(CHUNK // LANES):
                pos = c * CHUNK + j * LANES + lane
                s = pos // L
                l = pos - s * L
                tok = plsc.load_gather(seq_v, [s, l, zero16])
                w_idx[p, pl.ds(j * LANES, LANES)] = tok
                for i in range(R):
                    d = i - RAD
                    if d == 0:
                        ntok = tok
                    else:
                        lv = l + d
                        g = plsc.load_gather(
                            seq_v, [s, jnp.clip(lv, 0, L - 1), zero16]
                        )
                        valid = (lv >= 0) & (lv <= L - 1)
                        ntok = jnp.where(valid, g, 0)
                    u_idx[p, i, pl.ds(j * LANES, LANES)] = ntok * R + i
            pltpu.async_copy(W_hbm.at[w_idx.at[p]], w_rows.at[p], semw[p])
            for i in range(R):
                pltpu.async_copy(
                    U_hbm.at[u_idx.at[p, i]], u_rows.at[p, i], semu[p]
                )

        def drain(p):
            pltpu.make_async_copy(
                W_hbm.at[pl.ds(0, CHUNK)], w_rows.at[p], semw[p]
            ).wait()
            for i in range(R):
                pltpu.make_async_copy(
                    U_hbm.at[pl.ds(0, CHUNK)], u_rows.at[p, i], semu[p]
                ).wait()

        def compute(c, p):
            @pl.loop(0, CHUNK // LANES)
            def grp_loop(j):
                tok_vec = w_idx[p, pl.ds(j * LANES, LANES)]
                mvec = jnp.where(tok_vec != 0, 1.0, 0.0).astype(jnp.float32)
                for cl in range(LANES):
                    cc = j * LANES + cl
                    maskf = mvec[cl]
                    for e in range(E_SL):
                        es = pl.ds(e * LANES, LANES)
                        w_e = w_rows[p, cc, es]
                        acc = u_rows[p, 0, cc, es] * w_e
                        for i in range(1, R):
                            acc = jnp.maximum(acc, u_rows[p, i, cc, es] * w_e)
                        out_v[cc, es] = acc * maskf

            pltpu.sync_copy(out_v, out_hbm.at[pl.ds(base + c * CHUNK, CHUNK)])

        build_and_fire(0, 0)

        @pl.loop(0, (n_chunks - 1) // 2)
        def pair_loop(kk):
            c0 = 2 * kk
            build_and_fire(c0 + 1, 1)
            drain(0)
            compute(c0, 0)
            build_and_fire(c0 + 2, 0)
            drain(1)
            compute(c0 + 1, 1)

        drain(0)
        compute(n_chunks - 1, 0)

    return k(seq3d, W, U)


def kernel(seq, W, U):
    B, L, _ = seq.shape
    R = U.shape[0] // W.shape[0]
    out = _region_encode(seq, W, U, B=B, L=L, R=R)
    return out.reshape(B, L, 1, EMB)


# speedup vs baseline: 1.0208x; 1.0208x over previous
import functools

import jax
import jax.numpy as jnp
from jax import lax
from jax.experimental import pallas as pl
from jax.experimental.pallas import tpu as pltpu
from jax.experimental.pallas import tpu_sc as plsc

NC = 2
NS = 16
NW = NC * NS
LANES = 16

EMB = 64
E_SL = EMB // LANES
CHUNK = 64


def _flatten_seq(seq3d, *, B, L):
    BLK = 128

    def body(in_ref, out_ref):
        y = in_ref[...][:, :, 0]
        z = jnp.zeros((BLK, 128 - L), jnp.int32)
        out_ref[...] = jnp.concatenate([y, z], axis=1)

    return pl.pallas_call(
        body,
        grid=(B // BLK,),
        in_specs=[pl.BlockSpec((BLK, L, 1), lambda i: (i, 0, 0))],
        out_specs=pl.BlockSpec((BLK, 128), lambda i: (i, 0)),
        out_shape=jax.ShapeDtypeStruct((B, 128), jnp.int32),
    )(seq3d)


def _region_encode(seqp, W, U, *, B, L, R):
    TOK = B * L
    b_per_w = B // NW
    per_w = TOK // NW
    n_chunks = per_w // CHUNK
    RAD = (R - 1) // 2

    mesh = plsc.VectorSubcoreMesh(
        core_axis_name="c", subcore_axis_name="s", num_cores=NC, num_subcores=NS
    )

    @functools.partial(
        pl.kernel,
        out_type=jax.ShapeDtypeStruct((TOK, EMB), jnp.float32),
        mesh=mesh,
        compiler_params=pltpu.CompilerParams(
            needs_layout_passes=False, use_tc_tiling_on_sc=False
        ),
        scratch_types=[
            pltpu.VMEM((b_per_w, 128), jnp.int32),
            pltpu.VMEM((2, CHUNK), jnp.int32),
            pltpu.VMEM((2, R, CHUNK), jnp.int32),
            pltpu.VMEM((2, CHUNK, EMB), jnp.float32),
            pltpu.VMEM((2, R, CHUNK, EMB), jnp.float32),
            pltpu.VMEM((CHUNK, EMB), jnp.float32),
            pltpu.SemaphoreType.DMA,
            pltpu.SemaphoreType.DMA,
            pltpu.SemaphoreType.DMA,
            pltpu.SemaphoreType.DMA,
        ],
    )
    def k(seq_hbm, W_hbm, U_hbm, out_hbm,
          seq_v, w_idx, u_idx, w_rows, u_rows, out_v, semw0, semw1, semu0, semu1):
        wid = lax.axis_index("s") * NC + lax.axis_index("c")
        base = wid * per_w
        pltpu.sync_copy(seq_hbm.at[pl.ds(wid * b_per_w, b_per_w)], seq_v)

        lane = lax.broadcasted_iota(jnp.int32, (LANES,), 0)
        semw = (semw0, semw1)
        semu = (semu0, semu1)

        def build_and_fire(c, p):
            for j in range(CHUNK // LANES):
                pos = c * CHUNK + j * LANES + lane
                s = pos // L
                l = pos - s * L
                tok = plsc.load_gather(seq_v, [s, l])
                w_idx[p, pl.ds(j * LANES, LANES)] = tok
                for i in range(R):
                    d = i - RAD
                    if d == 0:
                        ntok = tok
                    else:
                        lv = l + d
                        g = plsc.load_gather(seq_v, [s, jnp.clip(lv, 0, L - 1)])
                        valid = (lv >= 0) & (lv <= L - 1)
                        ntok = jnp.where(valid, g, 0)
                    u_idx[p, i, pl.ds(j * LANES, LANES)] = ntok * R + i
            pltpu.async_copy(W_hbm.at[w_idx.at[p]], w_rows.at[p], semw[p])
            for i in range(R):
                pltpu.async_copy(
                    U_hbm.at[u_idx.at[p, i]], u_rows.at[p, i], semu[p]
                )

        def drain(p):
            pltpu.make_async_copy(
                W_hbm.at[pl.ds(0, CHUNK)], w_rows.at[p], semw[p]
            ).wait()
            for i in range(R):
                pltpu.make_async_copy(
                    U_hbm.at[pl.ds(0, CHUNK)], u_rows.at[p, i], semu[p]
                ).wait()

        def compute(c, p):
            @pl.loop(0, CHUNK // LANES)
            def grp_loop(j):
                tok_vec = w_idx[p, pl.ds(j * LANES, LANES)]
                mvec = jnp.where(tok_vec != 0, 1.0, 0.0).astype(jnp.float32)
                for cl in range(LANES):
                    cc = j * LANES + cl
                    maskf = mvec[cl]
                    for e in range(E_SL):
                        es = pl.ds(e * LANES, LANES)
                        w_e = w_rows[p, cc, es]
                        acc = u_rows[p, 0, cc, es] * w_e
                        for i in range(1, R):
                            acc = jnp.maximum(acc, u_rows[p, i, cc, es] * w_e)
                        out_v[cc, es] = acc * maskf

            pltpu.sync_copy(out_v, out_hbm.at[pl.ds(base + c * CHUNK, CHUNK)])

        build_and_fire(0, 0)

        @pl.loop(0, (n_chunks - 1) // 2)
        def pair_loop(kk):
            c0 = 2 * kk
            build_and_fire(c0 + 1, 1)
            drain(0)
            compute(c0, 0)
            build_and_fire(c0 + 2, 0)
            drain(1)
            compute(c0 + 1, 1)

        drain(0)
        compute(n_chunks - 1, 0)

    return k(seqp, W, U)


def kernel(seq, W, U):
    B, L, _ = seq.shape
    R = U.shape[0] // W.shape[0]
    seqp = _flatten_seq(seq, B=B, L=L)
    out = _region_encode(seqp, W, U, B=B, L=L, R=R)
    return out.reshape(B, L, 1, EMB)
